# Initial kernel scaffold; baseline (speedup 1.0000x reference)
import jax
import jax.numpy as jnp
from jax.experimental import pallas as pl


def kernel(x, table):
    raise NotImplementedError("write your pallas kernel here")


# speedup vs baseline: 1.1142x; 1.1142x over previous
import functools

import jax
import jax.numpy as jnp
from jax import lax
from jax.experimental import pallas as pl
from jax.experimental.pallas import tpu as pltpu
from jax.experimental.pallas import tpu_sc as plsc

CHUNK = 1600


def _build(N, V, D, n_per_w, num_cores):
    mesh = plsc.VectorSubcoreMesh(core_axis_name="c", subcore_axis_name="s")
    n_chunks = n_per_w // CHUNK

    @functools.partial(
        pl.kernel,
        mesh=mesh,
        out_type=jax.ShapeDtypeStruct((N, D), jnp.float32),
        scratch_types=[
            pltpu.VMEM((n_per_w,), jnp.int32),
            pltpu.VMEM((2, CHUNK, D), jnp.float32),
            pltpu.SemaphoreType.DMA,
            pltpu.SemaphoreType.DMA,
        ],
        compiler_params=pltpu.CompilerParams(use_tc_tiling_on_sc=False),
    )
    def k(x_hbm, table_hbm, out_hbm, idx_v, rows_v, gsem, osem):
        wid = lax.axis_index("s") * num_cores + lax.axis_index("c")
        base = wid * n_per_w

        pltpu.sync_copy(x_hbm.at[pl.ds(base, n_per_w)], idx_v)

        def gather(i):
            return pltpu.async_copy(
                table_hbm.at[idx_v.at[pl.ds(i * CHUNK, CHUNK)]],
                rows_v.at[i % 2],
                gsem,
            )

        def writeback(i):
            return pltpu.async_copy(
                rows_v.at[i % 2],
                out_hbm.at[pl.ds(base + i * CHUNK, CHUNK), :],
                osem,
            )

        gathers = [None] * n_chunks
        writes = [None] * n_chunks
        gathers[0] = gather(0)
        for i in range(n_chunks):
            if i >= 1:
                writes[i - 1].wait()
            if i + 1 < n_chunks:
                gathers[i + 1] = gather(i + 1)
            gathers[i].wait()
            writes[i] = writeback(i)
        writes[n_chunks - 1].wait()

    return k


def kernel(x, table):
    B, H = x.shape
    V, D = table.shape
    N = B * H
    info = plsc.get_sparse_core_info()
    nw = info.num_cores * info.num_subcores
    n_per_w = N // nw
    k = _build(N, V, D, n_per_w, info.num_cores)
    out = k(x.reshape(N), table)
    return out.reshape(B, H, D)
